# Initial kernel scaffold; baseline (speedup 1.0000x reference)
import jax
import jax.numpy as jnp
from jax.experimental import pallas as pl


def kernel(x, fe_W0, fe_b0, fe_W1, fe_b1, fn_W0, fn_b0, fn_W1, fn_b1):
    raise NotImplementedError("write your pallas kernel here")


# speedup vs baseline: 3.1561x; 3.1561x over previous
import jax
import jax.numpy as jnp
from jax.experimental import pallas as pl

_ALPHA = 0.2


def _lrelu(v):
    return jnp.where(v >= 0, v, _ALPHA * v)


def _mp_kernel(x_ref, w0a_ref, w0b_ref, w0d_ref, b0_ref, w1_ref, b1_ref,
               nw0_ref, nb0_ref, nw1_ref, nb1_ref, out_ref):
    x = x_ref[0]
    n = x.shape[0]

    u = jnp.dot(x, w0a_ref[...], preferred_element_type=jnp.float32)
    v = jnp.dot(x, w0b_ref[...], preferred_element_type=jnp.float32)
    v = v + b0_ref[...]

    diffs = x[None, :, :] - x[:, None, :] + 1e-12
    dist = jnp.sqrt(jnp.sum(diffs * diffs, axis=2))

    e = u[:, None, :] + v[None, :, :] + dist[:, :, None] * w0d_ref[...][None]
    e = _lrelu(e)

    e2 = jnp.dot(e.reshape(n * n, -1), w1_ref[...],
                 preferred_element_type=jnp.float32) + b1_ref[...]
    e2 = _lrelu(e2).reshape(n, n, -1)

    agg = jnp.sum(e2, axis=1)

    h = jnp.concatenate([agg, x], axis=1)
    h = _lrelu(jnp.dot(h, nw0_ref[...],
                       preferred_element_type=jnp.float32) + nb0_ref[...])
    h = jnp.dot(h, nw1_ref[...],
                preferred_element_type=jnp.float32) + nb1_ref[...]
    out_ref[0] = h


def kernel(x, fe_W0, fe_b0, fe_W1, fe_b1, fn_W0, fn_b0, fn_W1, fn_b1):
    B, N, D = x.shape
    F1 = fe_W0.shape[1]
    F2 = fe_W1.shape[1]
    FO = fn_W1.shape[1]

    w0a = fe_W0[:D]
    w0b = fe_W0[D:2 * D]
    w0d = fe_W0[2 * D:]

    full = lambda shape: pl.BlockSpec(shape, lambda b: (0,) * len(shape))

    return pl.pallas_call(
        _mp_kernel,
        grid=(B,),
        in_specs=[
            pl.BlockSpec((1, N, D), lambda b: (b, 0, 0)),
            full((D, F1)), full((D, F1)), full((1, F1)), full((1, F1)),
            full((F1, F2)), full((1, F2)),
            full((F2 + D, fn_W0.shape[1])), full((1, fn_W0.shape[1])),
            full((fn_W0.shape[1], FO)), full((1, FO)),
        ],
        out_specs=pl.BlockSpec((1, N, FO), lambda b: (b, 0, 0)),
        out_shape=jax.ShapeDtypeStruct((B, N, FO), jnp.float32),
    )(x, w0a, w0b, w0d, fe_b0.reshape(1, -1), fe_W1, fe_b1.reshape(1, -1),
      fn_W0, fn_b0.reshape(1, -1), fn_W1, fn_b1.reshape(1, -1))


# speedup vs baseline: 5.1839x; 1.6425x over previous
import jax
import jax.numpy as jnp
from jax.experimental import pallas as pl

_ALPHA = 0.2


def _lrelu(v):
    return jnp.maximum(v, _ALPHA * v)


def _mp_kernel(x_ref, w0a_ref, w0b_ref, w0d_ref, b0_ref, w1_ref, b1_ref,
               nw0_ref, nb0_ref, nw1_ref, nb1_ref, out_ref):
    x = x_ref[0]
    n = x.shape[0]

    u = jnp.dot(x, w0a_ref[...], preferred_element_type=jnp.float32)
    v = jnp.dot(x, w0b_ref[...], preferred_element_type=jnp.float32)
    v = v + b0_ref[...]

    xx = x * x
    g = jax.lax.dot_general(x, x, (((1,), (1,)), ((), ())),
                            preferred_element_type=jnp.float32)
    sq_col = jnp.sum(xx, axis=1, keepdims=True)
    ones_row = jnp.ones((1, x.shape[1]), jnp.float32)
    sq_row = jax.lax.dot_general(ones_row, xx, (((1,), (1,)), ((), ())),
                                 preferred_element_type=jnp.float32)
    dist = jnp.sqrt(jnp.maximum(sq_col + sq_row - 2.0 * g, 0.0))

    e = u[:, None, :] + v[None, :, :] + dist[:, :, None] * w0d_ref[...][None]
    e = _lrelu(e)

    e2 = jnp.dot(e.reshape(n * n, -1), w1_ref[...],
                 preferred_element_type=jnp.float32) + b1_ref[...]
    e2 = _lrelu(e2).reshape(n, n, -1)

    agg = jnp.sum(e2, axis=1)

    h = jnp.concatenate([agg, x], axis=1)
    h = _lrelu(jnp.dot(h, nw0_ref[...],
                       preferred_element_type=jnp.float32) + nb0_ref[...])
    h = jnp.dot(h, nw1_ref[...],
                preferred_element_type=jnp.float32) + nb1_ref[...]
    out_ref[0] = h


def kernel(x, fe_W0, fe_b0, fe_W1, fe_b1, fn_W0, fn_b0, fn_W1, fn_b1):
    B, N, D = x.shape
    F1 = fe_W0.shape[1]
    F2 = fe_W1.shape[1]
    FO = fn_W1.shape[1]

    w0a = fe_W0[:D]
    w0b = fe_W0[D:2 * D]
    w0d = fe_W0[2 * D:]

    full = lambda shape: pl.BlockSpec(shape, lambda b: (0,) * len(shape))

    return pl.pallas_call(
        _mp_kernel,
        grid=(B,),
        in_specs=[
            pl.BlockSpec((1, N, D), lambda b: (b, 0, 0)),
            full((D, F1)), full((D, F1)), full((1, F1)), full((1, F1)),
            full((F1, F2)), full((1, F2)),
            full((F2 + D, fn_W0.shape[1])), full((1, fn_W0.shape[1])),
            full((fn_W0.shape[1], FO)), full((1, FO)),
        ],
        out_specs=pl.BlockSpec((1, N, FO), lambda b: (b, 0, 0)),
        out_shape=jax.ShapeDtypeStruct((B, N, FO), jnp.float32),
    )(x, w0a, w0b, w0d, fe_b0.reshape(1, -1), fe_W1, fe_b1.reshape(1, -1),
      fn_W0, fn_b0.reshape(1, -1), fn_W1, fn_b1.reshape(1, -1))
